# initial kernel scaffold (unmeasured)
import jax
import jax.numpy as jnp
from jax import lax
from jax.experimental import pallas as pl
from jax.experimental.pallas import tpu as pltpu


def kernel(
    x,
):
    def body(*refs):
        pass

    out_shape = jax.ShapeDtypeStruct(..., jnp.float32)
    return pl.pallas_call(body, out_shape=out_shape)(...)


# baseline (device time: 1066589 ns/iter reference)
import jax
import jax.numpy as jnp
from jax import lax
from jax.experimental import pallas as pl
from jax.experimental.pallas import tpu as pltpu


def kernel(x):
    m, n_full = x.shape
    n = n_full // 2
    half = m // 2

    def body(x_hbm, out_hbm, local_sem, y_send_sem, y_recv_sem,
             f_send_sem, f_recv_sem):
        mx = lax.axis_index("x")
        my = lax.axis_index("y")

        barrier = pltpu.get_barrier_semaphore()
        pl.semaphore_signal(barrier, inc=1, device_id=(mx, 1 - my),
                            device_id_type=pl.DeviceIdType.MESH)
        pl.semaphore_signal(barrier, inc=1, device_id=(1 - mx, my),
                            device_id_type=pl.DeviceIdType.MESH)
        pl.semaphore_wait(barrier, 2)

        local = pltpu.make_async_copy(
            x_hbm.at[:, pl.ds(my * n, n)],
            out_hbm.at[pl.ds(my * m, m), :],
            local_sem,
        )
        local.start()

        y_rdma = pltpu.make_async_remote_copy(
            src_ref=x_hbm.at[pl.ds(mx * half, half), pl.ds((1 - my) * n, n)],
            dst_ref=out_hbm.at[pl.ds(my * m + mx * half, half), :],
            send_sem=y_send_sem,
            recv_sem=y_recv_sem,
            device_id=(mx, 1 - my),
            device_id_type=pl.DeviceIdType.MESH,
        )
        y_rdma.start()
        y_rdma.wait()

        fwd = pltpu.make_async_remote_copy(
            src_ref=out_hbm.at[pl.ds((1 - my) * m + mx * half, half), :],
            dst_ref=out_hbm.at[pl.ds((1 - my) * m + mx * half, half), :],
            send_sem=f_send_sem,
            recv_sem=f_recv_sem,
            device_id=(1 - mx, my),
            device_id_type=pl.DeviceIdType.MESH,
        )
        fwd.start()
        fwd.wait()
        local.wait()

    return pl.pallas_call(
        body,
        out_shape=jax.ShapeDtypeStruct((2 * m, n), x.dtype),
        in_specs=[pl.BlockSpec(memory_space=pl.ANY)],
        out_specs=pl.BlockSpec(memory_space=pl.ANY),
        scratch_shapes=[
            pltpu.SemaphoreType.DMA,
            pltpu.SemaphoreType.DMA,
            pltpu.SemaphoreType.DMA,
            pltpu.SemaphoreType.DMA,
            pltpu.SemaphoreType.DMA,
        ],
        compiler_params=pltpu.CompilerParams(collective_id=0),
    )(x)


# device time: 268670 ns/iter; 3.9699x vs baseline; 3.9699x over previous
import jax
import jax.numpy as jnp
from jax import lax
from jax.experimental import pallas as pl
from jax.experimental.pallas import tpu as pltpu

N_CHUNK = 16
L_ROWS = 512


def kernel(x):
    m, n_full = x.shape
    n = n_full // 2
    half = m // 2
    rows = half // N_CHUNK

    def body(x_hbm, out_hbm, y_send, y_recv, f_send, f_recv, vbuf, vsem):
        mx = lax.axis_index("x")
        my = lax.axis_index("y")

        barrier = pltpu.get_barrier_semaphore()
        pl.semaphore_signal(barrier, inc=1, device_id=(mx, 1 - my),
                            device_id_type=pl.DeviceIdType.MESH)
        pl.semaphore_signal(barrier, inc=1, device_id=(1 - mx, my),
                            device_id_type=pl.DeviceIdType.MESH)
        pl.semaphore_wait(barrier, 2)

        y_rdmas = []
        for c in range(N_CHUNK):
            y_rdma = pltpu.make_async_remote_copy(
                src_ref=x_hbm.at[pl.ds(mx * half + c * rows, rows),
                                 pl.ds((1 - my) * n, n)],
                dst_ref=out_hbm.at[pl.ds(my * m + mx * half + c * rows, rows), :],
                send_sem=y_send.at[c],
                recv_sem=y_recv.at[c],
                device_id=(mx, 1 - my),
                device_id_type=pl.DeviceIdType.MESH,
            )
            y_rdma.start()
            y_rdmas.append(y_rdma)

        stores = [None, None]
        for c in range(m // L_ROWS):
            slot = c % 2
            ld = pltpu.make_async_copy(
                x_hbm.at[pl.ds(c * L_ROWS, L_ROWS), pl.ds(my * n, n)],
                vbuf.at[slot],
                vsem.at[slot],
            )
            if stores[slot] is not None:
                stores[slot].wait()
            ld.start()
            ld.wait()
            st = pltpu.make_async_copy(
                vbuf.at[slot],
                out_hbm.at[pl.ds(my * m + c * L_ROWS, L_ROWS), :],
                vsem.at[2 + slot],
            )
            st.start()
            stores[slot] = st

        fwds = []
        for c in range(N_CHUNK):
            y_rdmas[c].wait_recv()
            r = (1 - my) * m + mx * half + c * rows
            fwd = pltpu.make_async_remote_copy(
                src_ref=out_hbm.at[pl.ds(r, rows), :],
                dst_ref=out_hbm.at[pl.ds(r, rows), :],
                send_sem=f_send.at[c],
                recv_sem=f_recv.at[c],
                device_id=(1 - mx, my),
                device_id_type=pl.DeviceIdType.MESH,
            )
            fwd.start()
            fwds.append(fwd)

        for c in range(N_CHUNK):
            y_rdmas[c].wait_send()
            fwds[c].wait()
        stores[0].wait()
        stores[1].wait()

    return pl.pallas_call(
        body,
        out_shape=jax.ShapeDtypeStruct((2 * m, n), x.dtype),
        in_specs=[pl.BlockSpec(memory_space=pl.ANY)],
        out_specs=pl.BlockSpec(memory_space=pl.ANY),
        scratch_shapes=[
            pltpu.SemaphoreType.DMA((N_CHUNK,)),
            pltpu.SemaphoreType.DMA((N_CHUNK,)),
            pltpu.SemaphoreType.DMA((N_CHUNK,)),
            pltpu.SemaphoreType.DMA((N_CHUNK,)),
            pltpu.VMEM((2, L_ROWS, n), x.dtype),
            pltpu.SemaphoreType.DMA((4,)),
        ],
        compiler_params=pltpu.CompilerParams(collective_id=0),
    )(x)


# device time: 242779 ns/iter; 4.3933x vs baseline; 1.1066x over previous
import jax
import jax.numpy as jnp
from jax import lax
from jax.experimental import pallas as pl
from jax.experimental.pallas import tpu as pltpu

N_CHUNK = 16
L_ROWS = 512


def kernel(x):
    m, n_full = x.shape
    n = n_full // 2
    half = m // 2
    rows = half // N_CHUNK

    def body(x_hbm, out_hbm, y_send, y_recv, f_send, f_recv, vbuf, vsem):
        mx = lax.axis_index("x")
        my = lax.axis_index("y")

        barrier = pltpu.get_barrier_semaphore()
        pl.semaphore_signal(barrier, inc=1, device_id=(mx, 1 - my),
                            device_id_type=pl.DeviceIdType.MESH)
        pl.semaphore_signal(barrier, inc=1, device_id=(1 - mx, my),
                            device_id_type=pl.DeviceIdType.MESH)
        pl.semaphore_wait(barrier, 2)

        y_rdmas = []
        for c in range(N_CHUNK):
            y_rdma = pltpu.make_async_remote_copy(
                src_ref=x_hbm.at[pl.ds(mx * half + c * rows, rows),
                                 pl.ds((1 - my) * n, n)],
                dst_ref=out_hbm.at[pl.ds(my * m + mx * half + c * rows, rows), :],
                send_sem=y_send.at[c],
                recv_sem=y_recv.at[c],
                device_id=(mx, 1 - my),
                device_id_type=pl.DeviceIdType.MESH,
            )
            y_rdma.start()
            y_rdmas.append(y_rdma)

        fwds = []
        stores = [None, None]
        for c in range(N_CHUNK):
            y_rdmas[c].wait_recv()
            r = (1 - my) * m + mx * half + c * rows
            fwd = pltpu.make_async_remote_copy(
                src_ref=out_hbm.at[pl.ds(r, rows), :],
                dst_ref=out_hbm.at[pl.ds(r, rows), :],
                send_sem=f_send.at[c],
                recv_sem=f_recv.at[c],
                device_id=(1 - mx, my),
                device_id_type=pl.DeviceIdType.MESH,
            )
            fwd.start()
            fwds.append(fwd)

            slot = c % 2
            ld = pltpu.make_async_copy(
                x_hbm.at[pl.ds(c * L_ROWS, L_ROWS), pl.ds(my * n, n)],
                vbuf.at[slot],
                vsem.at[slot],
            )
            if stores[slot] is not None:
                stores[slot].wait()
            ld.start()
            ld.wait()
            st = pltpu.make_async_copy(
                vbuf.at[slot],
                out_hbm.at[pl.ds(my * m + c * L_ROWS, L_ROWS), :],
                vsem.at[2 + slot],
            )
            st.start()
            stores[slot] = st

        for c in range(N_CHUNK):
            y_rdmas[c].wait_send()
            fwds[c].wait()
        stores[0].wait()
        stores[1].wait()

    return pl.pallas_call(
        body,
        out_shape=jax.ShapeDtypeStruct((2 * m, n), x.dtype),
        in_specs=[pl.BlockSpec(memory_space=pl.ANY)],
        out_specs=pl.BlockSpec(memory_space=pl.ANY),
        scratch_shapes=[
            pltpu.SemaphoreType.DMA((N_CHUNK,)),
            pltpu.SemaphoreType.DMA((N_CHUNK,)),
            pltpu.SemaphoreType.DMA((N_CHUNK,)),
            pltpu.SemaphoreType.DMA((N_CHUNK,)),
            pltpu.VMEM((2, L_ROWS, n), x.dtype),
            pltpu.SemaphoreType.DMA((4,)),
        ],
        compiler_params=pltpu.CompilerParams(collective_id=0),
    )(x)
